# baseline (speedup 1.0000x reference)
import functools

import jax
import jax.numpy as jnp
from jax import lax
from jax.experimental import pallas as pl
from jax.experimental.pallas import tpu as pltpu
from jax.experimental.pallas import tpu_sc as plsc

B = 1024
L = 200
H = 64
FLAT = B * L
NC = 2
NS = 16
NW = NC * NS
PER_W = FLAT // NW
IDX_MINOR = 100
IDX_ROWS = PER_W // IDX_MINOR
CHUNK = L
DMAS_PER_CHUNK = CHUNK // IDX_MINOR
NCHUNK = PER_W // CHUNK
LANES = 16


def _emb_body(x_hbm, tok_hbm, pos_hbm, out_hbm, idx_v, pos_v, rows_v, gsem):
    wid = lax.axis_index("s") * NC + lax.axis_index("c")
    pltpu.sync_copy(x_hbm.at[wid], idx_v)
    pltpu.sync_copy(pos_hbm.at[pl.ds(0, L)], pos_v)

    def chunk_body(c, carry):
        cps = []
        for h in range(DMAS_PER_CHUNK):
            cps.append(
                pltpu.async_copy(
                    tok_hbm.at[idx_v.at[c * DMAS_PER_CHUNK + h]],
                    rows_v.at[pl.ds(h * IDX_MINOR, IDX_MINOR)],
                    gsem,
                )
            )
        for cp in cps:
            cp.wait()

        def add_body(r, carry2):
            for k in range(H // LANES):
                s = pl.ds(k * LANES, LANES)
                rows_v[r, s] = rows_v[r, s] + pos_v[r, s]
            return carry2

        lax.fori_loop(0, CHUNK, add_body, 0)

        pltpu.sync_copy(rows_v, out_hbm.at[wid * NCHUNK + c])
        return carry

    lax.fori_loop(0, NCHUNK, chunk_body, 0)


@functools.cache
def _build_kernel():
    return functools.partial(
        pl.kernel,
        out_type=jax.ShapeDtypeStruct((B, L, H), jnp.float32),
        mesh=plsc.VectorSubcoreMesh(core_axis_name="c", subcore_axis_name="s"),
        scratch_types=[
            pltpu.VMEM((IDX_ROWS, IDX_MINOR), jnp.int32),
            pltpu.VMEM((L, H), jnp.float32),
            pltpu.VMEM((CHUNK, H), jnp.float32),
            pltpu.SemaphoreType.DMA,
        ],
        compiler_params=pltpu.CompilerParams(use_tc_tiling_on_sc=False),
    )(_emb_body)


def kernel(x, token_table, pos_table):
    x_flat = x.reshape(NW, IDX_ROWS, IDX_MINOR)
    return _build_kernel()(x_flat, token_table, pos_table)


# speedup vs baseline: 1.0599x; 1.0599x over previous
import functools

import jax
import jax.numpy as jnp
from jax import lax
from jax.experimental import pallas as pl
from jax.experimental.pallas import tpu as pltpu
from jax.experimental.pallas import tpu_sc as plsc

B = 1024
L = 200
H = 64
NC = 2
NS = 16
NW = NC * NS
BBLK = 128
NBBLK = B // BBLK
L_PER_W = L * NBBLK // NW
UNITS = L_PER_W
LANES = 16


def _add_write(out_hbm, pos_v, rows_v, l0, bblk, u):
    p = [pos_v[u, pl.ds(k * LANES, LANES)] for k in range(H // LANES)]

    def rbody(r, carry):
        for k in range(H // LANES):
            s = pl.ds(k * LANES, LANES)
            rows_v[r, s] = rows_v[r, s] + p[k]
        return carry

    lax.fori_loop(0, BBLK, rbody, 0, unroll=4)
    pltpu.sync_copy(
        rows_v,
        out_hbm.at[pl.ds((l0 + u) * B + bblk * BBLK, BBLK)],
    )


def _emb_body(xT_hbm, tok_hbm, pos_hbm, out_hbm,
              idx_v, pos_v, rows0, rows1, sem0, sem1):
    w = lax.axis_index("s") * NC + lax.axis_index("c")
    bblk = lax.rem(w, NBBLK)
    l0 = lax.div(w, NBBLK) * L_PER_W
    pltpu.sync_copy(
        xT_hbm.at[pl.ds(l0, L_PER_W), pl.ds(bblk * BBLK, BBLK)], idx_v
    )
    pltpu.sync_copy(pos_hbm.at[pl.ds(l0, L_PER_W)], pos_v)

    pltpu.async_copy(tok_hbm.at[idx_v.at[0]], rows0, sem0)

    def pair(j, carry):
        a = 2 * j
        b = a + 1
        pltpu.async_copy(tok_hbm.at[idx_v.at[b]], rows1, sem1)
        pltpu.make_async_copy(tok_hbm.at[idx_v.at[a]], rows0, sem0).wait()
        _add_write(out_hbm, pos_v, rows0, l0, bblk, a)
        nxt = jnp.minimum(a + 2, UNITS - 1)
        pltpu.async_copy(tok_hbm.at[idx_v.at[nxt]], rows0, sem0)
        pltpu.make_async_copy(tok_hbm.at[idx_v.at[b]], rows1, sem1).wait()
        _add_write(out_hbm, pos_v, rows1, l0, bblk, b)
        return carry

    lax.fori_loop(0, UNITS // 2, pair, 0)
    pltpu.make_async_copy(tok_hbm.at[idx_v.at[0]], rows0, sem0).wait()


@functools.cache
def _build_kernel():
    return functools.partial(
        pl.kernel,
        out_type=jax.ShapeDtypeStruct((L * B, H), jnp.float32),
        mesh=plsc.VectorSubcoreMesh(core_axis_name="c", subcore_axis_name="s"),
        scratch_types=[
            pltpu.VMEM((L_PER_W, BBLK), jnp.int32),
            pltpu.VMEM((L_PER_W, H), jnp.float32),
            pltpu.VMEM((BBLK, H), jnp.float32),
            pltpu.VMEM((BBLK, H), jnp.float32),
            pltpu.SemaphoreType.DMA,
            pltpu.SemaphoreType.DMA,
        ],
        compiler_params=pltpu.CompilerParams(use_tc_tiling_on_sc=False),
    )(_emb_body)


def kernel(x, token_table, pos_table):
    out = _build_kernel()(x.T, token_table, pos_table)
    return out.reshape(L, B, H).transpose(1, 0, 2)
